# baseline (speedup 1.0000x reference)
import functools

import jax
import jax.numpy as jnp
from jax import lax
from jax.experimental import pallas as pl
from jax.experimental.pallas import tpu as pltpu
from jax.experimental.pallas import tpu_sc as plsc

N = 16384
C = 1000
NC = 2
NS = 16
L = 16
NW = NC * NS
RW = N // NW
NGRP = RW // L

CR = 200
NCR = C // CR


def _argmax_body(x_ref, o_ref, m_ref):
    k = pl.program_id(0)
    x = x_ref[...]
    rows = lax.broadcasted_iota(jnp.int32, x.shape, 0) + k * CR
    m = jnp.max(x, axis=0)
    idx = jnp.min(jnp.where(x == m[None, :], rows, jnp.int32(C)), axis=0)

    @pl.when(k == 0)
    def _():
        m_ref[...] = m
        o_ref[...] = idx

    @pl.when(k > 0)
    def _():
        better = m > m_ref[...]
        o_ref[...] = jnp.where(better, idx, o_ref[...])
        m_ref[...] = jnp.maximum(m, m_ref[...])


def _argmax(cp_t):
    return pl.pallas_call(
        _argmax_body,
        grid=(NCR,),
        in_specs=[pl.BlockSpec((CR, N), lambda k: (k, 0))],
        out_specs=pl.BlockSpec((N,), lambda k: (0,)),
        out_shape=jax.ShapeDtypeStruct((N,), jnp.int32),
        scratch_shapes=[pltpu.VMEM((N,), jnp.float32)],
    )(cp_t)


def _affine_body(cls_hbm, three_hbm, tr_hbm, sc_hbm, out_hbm,
                 cls_buf, tr_buf, sc_buf, three_buf, out_buf, lsem, ssem):
    cid = lax.axis_index("c")
    sid = lax.axis_index("s")
    wid = sid * NC + cid
    base = wid * RW

    loads = [
        pltpu.async_copy(tr_hbm, tr_buf, lsem),
        pltpu.async_copy(sc_hbm, sc_buf, lsem),
        pltpu.async_copy(cls_hbm.at[pl.ds(base, RW)], cls_buf, lsem),
    ] + [
        pltpu.async_copy(three_hbm.at[pl.ds(d * N + base, RW)],
                         three_buf.at[pl.ds(d * RW, RW)], lsem)
        for d in range(3)
    ]
    for cp in loads:
        cp.wait()

    for g in range(NGRP):
        cls16 = cls_buf[pl.ds(g * L, L)]
        for d in range(3):
            tr = plsc.load_gather(tr_buf, [cls16 + d * C])
            sc = plsc.load_gather(sc_buf, [cls16 + d * C])
            th = three_buf[pl.ds(d * RW + g * L, L)]
            out_buf[pl.ds(d * RW + g * L, L)] = th * sc + tr

    stores = [
        pltpu.async_copy(out_buf.at[pl.ds(d * RW, RW)],
                         out_hbm.at[pl.ds(d * N + base, RW)], ssem)
        for d in range(3)
    ]
    for cp in stores:
        cp.wait()


def _affine(classes, three_flat, tr_flat, sc_flat):
    mesh = plsc.VectorSubcoreMesh(core_axis_name="c", subcore_axis_name="s")
    f = functools.partial(
        pl.kernel,
        out_type=jax.ShapeDtypeStruct((3 * N,), jnp.float32),
        mesh=mesh,
        scratch_types=[
            pltpu.VMEM((RW,), jnp.int32),
            pltpu.VMEM((3 * C,), jnp.float32),
            pltpu.VMEM((3 * C,), jnp.float32),
            pltpu.VMEM((3 * RW,), jnp.float32),
            pltpu.VMEM((3 * RW,), jnp.float32),
            pltpu.SemaphoreType.DMA,
            pltpu.SemaphoreType.DMA,
        ],
        compiler_params=pltpu.CompilerParams(needs_layout_passes=False),
    )(_affine_body)
    return f(classes, three_flat, tr_flat, sc_flat)


def kernel(class_pred, three_pred, geo_dict, translation, scale):
    del geo_dict
    classes = _argmax(class_pred.T)
    out_flat = _affine(classes, three_pred.T.reshape(3 * N),
                       translation.reshape(3 * C), scale.reshape(3 * C))
    return out_flat.reshape(3, N).T


# speedup vs baseline: 1.0638x; 1.0638x over previous
import functools

import jax
import jax.numpy as jnp
from jax import lax
from jax.experimental import pallas as pl
from jax.experimental.pallas import tpu as pltpu
from jax.experimental.pallas import tpu_sc as plsc

N = 16384
C = 1000
NC = 2
NS = 16
L = 16
NW = NC * NS
RW = N // NW
NGRP = RW // L

BC = 4096
NBC = N // BC


def _argmax_body(x_ref, o_ref):
    x = x_ref[...]
    m = jnp.max(x, axis=0)
    rows = lax.broadcasted_iota(jnp.int32, x.shape, 0)
    idx = jnp.min(jnp.where(x == m[None, :], rows, jnp.int32(C)), axis=0)
    o_ref[...] = idx


def _argmax(cp_t):
    return pl.pallas_call(
        _argmax_body,
        grid=(NBC,),
        in_specs=[pl.BlockSpec((C, BC), lambda k: (0, k))],
        out_specs=pl.BlockSpec((BC,), lambda k: (k,)),
        out_shape=jax.ShapeDtypeStruct((N,), jnp.int32),
    )(cp_t)


def _affine_body(cls_hbm, three_hbm, tr_hbm, sc_hbm, out_hbm,
                 cls_buf, tr_buf, sc_buf, three_buf, out_buf, lsem, ssem):
    cid = lax.axis_index("c")
    sid = lax.axis_index("s")
    wid = sid * NC + cid
    base = wid * RW

    loads = [
        pltpu.async_copy(tr_hbm, tr_buf, lsem),
        pltpu.async_copy(sc_hbm, sc_buf, lsem),
        pltpu.async_copy(cls_hbm.at[pl.ds(base, RW)], cls_buf, lsem),
    ] + [
        pltpu.async_copy(three_hbm.at[pl.ds(d * N + base, RW)],
                         three_buf.at[pl.ds(d * RW, RW)], lsem)
        for d in range(3)
    ]
    for cp in loads:
        cp.wait()

    for g in range(NGRP):
        cls16 = cls_buf[pl.ds(g * L, L)]
        for d in range(3):
            tr = plsc.load_gather(tr_buf, [cls16 + d * C])
            sc = plsc.load_gather(sc_buf, [cls16 + d * C])
            th = three_buf[pl.ds(d * RW + g * L, L)]
            out_buf[pl.ds(d * RW + g * L, L)] = th * sc + tr

    stores = [
        pltpu.async_copy(out_buf.at[pl.ds(d * RW, RW)],
                         out_hbm.at[pl.ds(d * N + base, RW)], ssem)
        for d in range(3)
    ]
    for cp in stores:
        cp.wait()


def _affine(classes, three_flat, tr_flat, sc_flat):
    mesh = plsc.VectorSubcoreMesh(core_axis_name="c", subcore_axis_name="s")
    f = functools.partial(
        pl.kernel,
        out_type=jax.ShapeDtypeStruct((3 * N,), jnp.float32),
        mesh=mesh,
        scratch_types=[
            pltpu.VMEM((RW,), jnp.int32),
            pltpu.VMEM((3 * C,), jnp.float32),
            pltpu.VMEM((3 * C,), jnp.float32),
            pltpu.VMEM((3 * RW,), jnp.float32),
            pltpu.VMEM((3 * RW,), jnp.float32),
            pltpu.SemaphoreType.DMA,
            pltpu.SemaphoreType.DMA,
        ],
        compiler_params=pltpu.CompilerParams(needs_layout_passes=False),
    )(_affine_body)
    return f(classes, three_flat, tr_flat, sc_flat)


def kernel(class_pred, three_pred, geo_dict, translation, scale):
    del geo_dict
    classes = _argmax(class_pred.T)
    out_flat = _affine(classes, three_pred.T.reshape(3 * N),
                       translation.reshape(3 * C), scale.reshape(3 * C))
    return out_flat.reshape(3, N).T
